# Initial kernel scaffold; baseline (speedup 1.0000x reference)
import functools

import jax
import jax.numpy as jnp
from jax import lax
from jax.experimental import pallas as pl
from jax.experimental.pallas import tpu as pltpu
from jax.experimental.pallas import tpu_sc as plsc

L = 2048
D_MODEL = 768
D_CKV = 192
D_CQ = 256
N_HEAD = 8
D_HEAD = 64
D_ROPE = 32
HALF = D_ROPE // 2
K_TS = 32
WINDOW = 16
N_IDX = 2
ROPE_BASE = 10000.0
SCALE = (D_HEAD + D_ROPE) ** (-0.5)

_F32 = jnp.float32
_NEG_SENTINEL = jnp.int32(-(2 ** 31))


def _softplus(x):
    return jnp.maximum(x, 0.0) + jnp.log1p(jnp.exp(-jnp.abs(x)))


def _precompute_body(x_ref, w_dkv_ref, w_kr_ref, w_dq_ref, w_uq_ref, w_qr_ref,
                     w_uk_ref, idx_wq_ref, idx_wk_ref, qcos_ref, qsin_ref,
                     ckvsp_ref, qabs_ref, qrope_ref, qi_ref, ki_ref):
    x = x_ref[...]
    ckvsp_ref[:, :D_CKV] = jnp.dot(x, w_dkv_ref[...], preferred_element_type=_F32)
    ckvsp_ref[:, D_CKV:] = _softplus(
        jnp.dot(x, w_kr_ref[...], preferred_element_type=_F32))
    c_q = jnp.dot(x, w_dq_ref[...], preferred_element_type=_F32)
    q_c = jnp.dot(c_q, w_uq_ref[...], preferred_element_type=_F32)
    for h in range(N_HEAD):
        qc_h = q_c[:, h * D_HEAD:(h + 1) * D_HEAD]
        w_uk_h = w_uk_ref[:, h * D_HEAD:(h + 1) * D_HEAD]
        qabs_ref[:, h * D_CKV:(h + 1) * D_CKV] = lax.dot_general(
            qc_h, w_uk_h, (((1,), (1,)), ((), ())),
            preferred_element_type=_F32)
    sp_q = _softplus(jnp.dot(c_q, w_qr_ref[...], preferred_element_type=_F32))
    qcos = qcos_ref[...]
    qsin = qsin_ref[...]
    for h in range(N_HEAD):
        mu1 = sp_q[:, h * D_ROPE:h * D_ROPE + HALF]
        mu2 = sp_q[:, h * D_ROPE + HALF:(h + 1) * D_ROPE]
        qrope_ref[:, h * D_ROPE:h * D_ROPE + HALF] = mu1 * qcos - mu2 * qsin
        qrope_ref[:, h * D_ROPE + HALF:(h + 1) * D_ROPE] = mu1 * qsin + mu2 * qcos
    qi_ref[...] = jnp.dot(x, idx_wq_ref[...], preferred_element_type=_F32)
    ki_ref[...] = jnp.dot(x, idx_wk_ref[...], preferred_element_type=_F32)


def _precompute(x2, w_dkv, w_kr, w_dq, w_uq, w_qr, w_uk, idx_wq, idx_wk,
                qcos, qsin):
    B = 256
    grid = (L // B,)
    full = lambda shape: pl.BlockSpec(shape, lambda i: (0, 0))
    blk = lambda cols: pl.BlockSpec((B, cols), lambda i: (i, 0))
    return pl.pallas_call(
        _precompute_body,
        grid=grid,
        in_specs=[
            blk(D_MODEL),
            full((D_MODEL, D_CKV)),
            full((D_MODEL, D_ROPE)),
            full((D_MODEL, D_CQ)),
            full((D_CQ, N_HEAD * D_HEAD)),
            full((D_CQ, N_HEAD * D_ROPE)),
            full((D_CKV, N_HEAD * D_HEAD)),
            full((D_MODEL, N_IDX * D_HEAD)),
            full((D_MODEL, D_HEAD)),
            blk(HALF),
            blk(HALF),
        ],
        out_specs=[
            blk(D_CKV + D_ROPE),
            blk(N_HEAD * D_CKV),
            blk(N_HEAD * D_ROPE),
            blk(N_IDX * D_HEAD),
            blk(D_HEAD),
        ],
        out_shape=[
            jax.ShapeDtypeStruct((L, D_CKV + D_ROPE), _F32),
            jax.ShapeDtypeStruct((L, N_HEAD * D_CKV), _F32),
            jax.ShapeDtypeStruct((L, N_HEAD * D_ROPE), _F32),
            jax.ShapeDtypeStruct((L, N_IDX * D_HEAD), _F32),
            jax.ShapeDtypeStruct((L, D_HEAD), _F32),
        ],
    )(x2, w_dkv, w_kr, w_dq, w_uq, w_qr, w_uk, idx_wq, idx_wk, qcos, qsin)


def _topk_body(qi_ref, ki_ref, idxw_ref, idx_ref, keys_ref):
    blk = pl.program_id(0)
    B = qi_ref.shape[0]
    ki = ki_ref[...]
    acc = None
    for h in range(N_IDX):
        qi_h = qi_ref[:, h * D_HEAD:(h + 1) * D_HEAD]
        s = lax.dot_general(qi_h, ki, (((1,), (1,)), ((), ())),
                            preferred_element_type=_F32)
        s = jnp.maximum(s, 0.0) * idxw_ref[0, h]
        acc = s if acc is None else acc + s
    acc = jnp.where(acc == 0.0, 0.0, acc)
    row = blk * B + lax.broadcasted_iota(jnp.int32, (B, L), 0)
    col = lax.broadcasted_iota(jnp.int32, (B, L), 1)
    local = (col >= row - (WINDOW - 1)) & (col <= row)
    future = col > row
    vals = jnp.where(local, jnp.inf, jnp.where(future, -jnp.inf, acc))
    keys_ref[...] = lax.bitcast_convert_type(vals, jnp.int32)
    cols = []
    for _ in range(K_TS):
        k = keys_ref[...]
        m = jnp.max(k, axis=1, keepdims=True)
        cand = jnp.where(k == m, col, jnp.int32(L))
        amin = jnp.min(cand, axis=1)
        cols.append(amin.reshape(B, 1))
        keys_ref[...] = jnp.where(col == amin[:, None], _NEG_SENTINEL, k)
    idx_ref[...] = jnp.concatenate(cols, axis=1)


def _topk(qi, ki, idx_w2):
    B = 256
    return pl.pallas_call(
        _topk_body,
        grid=(L // B,),
        in_specs=[
            pl.BlockSpec((B, N_IDX * D_HEAD), lambda i: (i, 0)),
            pl.BlockSpec((L, D_HEAD), lambda i: (0, 0)),
            pl.BlockSpec(memory_space=pltpu.SMEM),
        ],
        out_specs=pl.BlockSpec((B, K_TS), lambda i: (i, 0)),
        out_shape=jax.ShapeDtypeStruct((L, K_TS), jnp.int32),
        scratch_shapes=[pltpu.VMEM((B, L), jnp.int32)],
    )(qi, ki, idx_w2)


def _sc_gather(table, idx_flat):
    n_rows = L * K_TS
    d = D_CKV + D_ROPE
    info = plsc.get_sparse_core_info()
    nw = info.num_cores * info.num_subcores
    chunk = 128
    per_w = n_rows // nw
    n_chunks = per_w // chunk
    mesh = plsc.VectorSubcoreMesh(core_axis_name="c", subcore_axis_name="s")

    @functools.partial(
        pl.kernel,
        mesh=mesh,
        out_type=jax.ShapeDtypeStruct((n_rows, d), _F32),
        scratch_types=[
            pltpu.VMEM((chunk,), jnp.int32),
            pltpu.VMEM((chunk, d), _F32),
            pltpu.SemaphoreType.DMA,
        ],
    )
    def gather_k(table_hbm, idx_hbm, out_hbm, idx_v, rows_v, sem):
        wid = lax.axis_index("s") * info.num_cores + lax.axis_index("c")

        def body(i, _):
            base = wid * per_w + i * chunk
            pltpu.sync_copy(idx_hbm.at[pl.ds(base, chunk)], idx_v)
            pltpu.async_copy(table_hbm.at[idx_v], rows_v, sem).wait()
            pltpu.sync_copy(rows_v, out_hbm.at[pl.ds(base, chunk)])
            return 0

        lax.fori_loop(0, n_chunks, body, 0)

    return gather_k(table, idx_flat)


def _attn_body(qabs_ref, qrope_ref, gath_ref, scos_ref, ssin_ref,
               w_uv_ref, w_out_ref, out_ref):
    B = qabs_ref.shape[0]
    gath = gath_ref[...]
    ckv = gath[:, :D_CKV].reshape(B, K_TS, D_CKV)
    sp = gath[:, D_CKV:].reshape(B, K_TS, D_ROPE)
    mu1 = sp[:, :, :HALF]
    mu2 = sp[:, :, HALF:]
    scos = scos_ref[...][None, :, :]
    ssin = ssin_ref[...][None, :, :]
    kr1 = mu1 * scos - mu2 * ssin
    kr2 = mu1 * ssin + mu2 * scos
    acc = jnp.zeros((B, D_MODEL), dtype=_F32)
    for h in range(N_HEAD):
        qa = qabs_ref[:, h * D_CKV:(h + 1) * D_CKV][:, None, :]
        sc = jnp.sum(qa * ckv, axis=-1)
        qr = qrope_ref[:, h * D_ROPE:(h + 1) * D_ROPE]
        qr1 = qr[:, :HALF][:, None, :]
        qr2 = qr[:, HALF:][:, None, :]
        sr = jnp.sum(qr1 * kr1 + qr2 * kr2, axis=-1)
        s = (sc + sr) * SCALE
        p = jnp.exp(s - jnp.max(s, axis=1, keepdims=True))
        a = p / jnp.sum(p, axis=1, keepdims=True)
        lat = jnp.sum(a[:, :, None] * ckv, axis=1)
        ho = jnp.dot(lat, w_uv_ref[:, h * D_HEAD:(h + 1) * D_HEAD],
                     preferred_element_type=_F32)
        acc = acc + jnp.dot(ho, w_out_ref[h * D_HEAD:(h + 1) * D_HEAD, :],
                            preferred_element_type=_F32)
    out_ref[...] = acc


def _attention(qabs, qrope, gath, scos, ssin, w_uv, w_out):
    B = 128
    d = D_CKV + D_ROPE
    return pl.pallas_call(
        _attn_body,
        grid=(L // B,),
        in_specs=[
            pl.BlockSpec((B, N_HEAD * D_CKV), lambda i: (i, 0)),
            pl.BlockSpec((B, N_HEAD * D_ROPE), lambda i: (i, 0)),
            pl.BlockSpec((B * K_TS, d), lambda i: (i, 0)),
            pl.BlockSpec((K_TS, HALF), lambda i: (0, 0)),
            pl.BlockSpec((K_TS, HALF), lambda i: (0, 0)),
            pl.BlockSpec((D_CKV, N_HEAD * D_HEAD), lambda i: (0, 0)),
            pl.BlockSpec((N_HEAD * D_HEAD, D_MODEL), lambda i: (0, 0)),
        ],
        out_specs=pl.BlockSpec((B, D_MODEL), lambda i: (i, 0)),
        out_shape=jax.ShapeDtypeStruct((L, D_MODEL), _F32),
    )(qabs, qrope, gath, scos, ssin, w_uv, w_out)


def kernel(x, w_dkv, w_uk, w_uv, w_dq, w_uq, w_qr, w_kr, raw_delta, w_out,
           idx_wq, idx_wk, idx_w):
    b = x.shape[0]
    x2 = x.reshape(L, D_MODEL)
    theta = 1.0 / (ROPE_BASE ** (2.0 * jnp.arange(HALF, dtype=_F32) / D_ROPE))
    delta = -2.0 * jnp.pi * jax.nn.sigmoid(raw_delta)
    ang_q = jnp.arange(L, dtype=_F32)[:, None] * theta[None, :] + delta[None, :]
    ang_s = jnp.arange(K_TS, dtype=_F32)[:, None] * theta[None, :] + delta[None, :]
    qcos, qsin = jnp.cos(ang_q), jnp.sin(ang_q)
    scos, ssin = jnp.cos(ang_s), jnp.sin(ang_s)

    ckvsp, qabs, qrope, qi, ki = _precompute(
        x2, w_dkv, w_kr, w_dq, w_uq, w_qr, w_uk, idx_wq, idx_wk, qcos, qsin)
    idx = _topk(qi, ki, idx_w.reshape(1, N_IDX))
    gath = _sc_gather(ckvsp, idx.reshape(-1))
    out2 = _attention(qabs, qrope, gath, scos, ssin, w_uv, w_out)
    return out2.reshape(b, L, D_MODEL)


# speedup vs baseline: 1.3864x; 1.3864x over previous
import functools

import jax
import jax.numpy as jnp
import numpy as np
from jax import lax
from jax.experimental import pallas as pl
from jax.experimental.pallas import tpu as pltpu
from jax.experimental.pallas import tpu_sc as plsc

L = 2048
D_MODEL = 768
D_CKV = 192
D_CQ = 256
N_HEAD = 8
D_HEAD = 64
D_ROPE = 32
HALF = D_ROPE // 2
K_TS = 32
WINDOW = 16
N_IDX = 2
ROPE_BASE = 10000.0
SCALE = (D_HEAD + D_ROPE) ** (-0.5)

D_TAB = 256

_F32 = jnp.float32
_NEG_SENTINEL = np.int32(-(2 ** 31))


def _softplus(x):
    return jnp.maximum(x, 0.0) + jnp.log1p(jnp.exp(-jnp.abs(x)))


def _precompute_body(x_ref, w_dkv_ref, w_kr_ref, w_dq_ref, w_uq_ref, w_qr_ref,
                     w_uk_ref, idx_wq_ref, idx_wk_ref, qcos_ref, qsin_ref,
                     ckvsp_ref, qabs_ref, qrope_ref, qi_ref, ki_ref):
    x = x_ref[...]
    ckvsp_ref[:, :D_CKV] = jnp.dot(x, w_dkv_ref[...], preferred_element_type=_F32)
    ckvsp_ref[:, D_CKV:D_CKV + D_ROPE] = _softplus(
        jnp.dot(x, w_kr_ref[...], preferred_element_type=_F32))
    ckvsp_ref[:, D_CKV + D_ROPE:] = jnp.zeros(
        (x.shape[0], D_TAB - D_CKV - D_ROPE), dtype=_F32)
    c_q = jnp.dot(x, w_dq_ref[...], preferred_element_type=_F32)
    q_c = jnp.dot(c_q, w_uq_ref[...], preferred_element_type=_F32)
    for h in range(N_HEAD):
        qc_h = q_c[:, h * D_HEAD:(h + 1) * D_HEAD]
        w_uk_h = w_uk_ref[:, h * D_HEAD:(h + 1) * D_HEAD]
        qabs_ref[:, h * D_CKV:(h + 1) * D_CKV] = lax.dot_general(
            qc_h, w_uk_h, (((1,), (1,)), ((), ())),
            preferred_element_type=_F32)
    sp_q = _softplus(jnp.dot(c_q, w_qr_ref[...], preferred_element_type=_F32))
    qcos = qcos_ref[...]
    qsin = qsin_ref[...]
    for h in range(N_HEAD):
        mu1 = sp_q[:, h * D_ROPE:h * D_ROPE + HALF]
        mu2 = sp_q[:, h * D_ROPE + HALF:(h + 1) * D_ROPE]
        qrope_ref[:, h * D_ROPE:h * D_ROPE + HALF] = mu1 * qcos - mu2 * qsin
        qrope_ref[:, h * D_ROPE + HALF:(h + 1) * D_ROPE] = mu1 * qsin + mu2 * qcos
    qi_ref[...] = jnp.dot(x, idx_wq_ref[...], preferred_element_type=_F32)
    ki_ref[...] = jnp.dot(x, idx_wk_ref[...], preferred_element_type=_F32)


def _precompute(x2, w_dkv, w_kr, w_dq, w_uq, w_qr, w_uk, idx_wq, idx_wk,
                qcos, qsin):
    B = 256
    grid = (L // B,)
    full = lambda shape: pl.BlockSpec(shape, lambda i: (0, 0))
    blk = lambda cols: pl.BlockSpec((B, cols), lambda i: (i, 0))
    return pl.pallas_call(
        _precompute_body,
        grid=grid,
        in_specs=[
            blk(D_MODEL),
            full((D_MODEL, D_CKV)),
            full((D_MODEL, D_ROPE)),
            full((D_MODEL, D_CQ)),
            full((D_CQ, N_HEAD * D_HEAD)),
            full((D_CQ, N_HEAD * D_ROPE)),
            full((D_CKV, N_HEAD * D_HEAD)),
            full((D_MODEL, N_IDX * D_HEAD)),
            full((D_MODEL, D_HEAD)),
            blk(HALF),
            blk(HALF),
        ],
        out_specs=[
            blk(D_TAB),
            blk(N_HEAD * D_CKV),
            blk(N_HEAD * D_ROPE),
            blk(N_IDX * D_HEAD),
            blk(D_HEAD),
        ],
        out_shape=[
            jax.ShapeDtypeStruct((L, D_TAB), _F32),
            jax.ShapeDtypeStruct((L, N_HEAD * D_CKV), _F32),
            jax.ShapeDtypeStruct((L, N_HEAD * D_ROPE), _F32),
            jax.ShapeDtypeStruct((L, N_IDX * D_HEAD), _F32),
            jax.ShapeDtypeStruct((L, D_HEAD), _F32),
        ],
    )(x2, w_dkv, w_kr, w_dq, w_uq, w_qr, w_uk, idx_wq, idx_wk, qcos, qsin)


def _topk_body(qi_ref, ki_ref, idxw_ref, idx_ref, keys_ref):
    blk = pl.program_id(0)
    B = qi_ref.shape[0]
    ki = ki_ref[...]
    acc = None
    for h in range(N_IDX):
        qi_h = qi_ref[:, h * D_HEAD:(h + 1) * D_HEAD]
        s = lax.dot_general(qi_h, ki, (((1,), (1,)), ((), ())),
                            preferred_element_type=_F32)
        s = s.astype(jnp.bfloat16).astype(_F32)
        s = jnp.maximum(s, 0.0) * idxw_ref[0, h]
        acc = s if acc is None else acc + s
    acc = jnp.where(acc == 0.0, 0.0, acc)
    row = blk * B + lax.broadcasted_iota(jnp.int32, (B, L), 0)
    col = lax.broadcasted_iota(jnp.int32, (B, L), 1)
    local = (col >= row - (WINDOW - 1)) & (col <= row)
    future = col > row
    vals = jnp.where(local, jnp.inf, jnp.where(future, -jnp.inf, acc))
    keys_ref[...] = lax.bitcast_convert_type(vals, jnp.int32)
    cols = []
    for _ in range(K_TS):
        k = keys_ref[...]
        m = jnp.max(k, axis=1, keepdims=True)
        cand = jnp.where(k == m, col, jnp.int32(L))
        amin = jnp.min(cand, axis=1)
        cols.append(amin.reshape(B, 1))
        keys_ref[...] = jnp.where(col == amin[:, None], _NEG_SENTINEL, k)
    idx_ref[...] = jnp.concatenate(cols, axis=1)


def _topk(qi, ki, idx_w2):
    B = 256
    return pl.pallas_call(
        _topk_body,
        grid=(L // B,),
        in_specs=[
            pl.BlockSpec((B, N_IDX * D_HEAD), lambda i: (i, 0)),
            pl.BlockSpec((L, D_HEAD), lambda i: (0, 0)),
            pl.BlockSpec(memory_space=pltpu.SMEM),
        ],
        out_specs=pl.BlockSpec((B, K_TS), lambda i: (i, 0)),
        out_shape=jax.ShapeDtypeStruct((L, K_TS), jnp.int32),
        scratch_shapes=[pltpu.VMEM((B, L), jnp.int32)],
    )(qi, ki, idx_w2)


def _sc_gather(table, idx_flat):
    n_rows = L * K_TS
    d = D_TAB
    info = plsc.get_sparse_core_info()
    nw = info.num_cores * info.num_subcores
    chunk = 128
    per_w = n_rows // nw
    n_chunks = per_w // chunk
    mesh = plsc.VectorSubcoreMesh(core_axis_name="c", subcore_axis_name="s")

    @functools.partial(
        pl.kernel,
        mesh=mesh,
        out_type=jax.ShapeDtypeStruct((n_rows, d), _F32),
        scratch_types=[
            pltpu.VMEM((chunk,), jnp.int32),
            pltpu.VMEM((chunk, d), _F32),
            pltpu.SemaphoreType.DMA,
        ],
    )
    def gather_k(table_hbm, idx_hbm, out_hbm, idx_v, rows_v, sem):
        wid = lax.axis_index("s") * info.num_cores + lax.axis_index("c")

        def body(i, _):
            base = wid * per_w + i * chunk
            pltpu.sync_copy(idx_hbm.at[pl.ds(base, chunk)], idx_v)
            pltpu.async_copy(table_hbm.at[idx_v], rows_v, sem).wait()
            pltpu.sync_copy(rows_v, out_hbm.at[pl.ds(base, chunk)])
            return 0

        lax.fori_loop(0, n_chunks, body, 0)

    return gather_k(table, idx_flat)


def _attn_body(qabs_ref, qrope_ref, gath_ref, scos_ref, ssin_ref,
               w_uv_ref, w_out_ref, out_ref):
    B = qabs_ref.shape[0]
    gath = gath_ref[...]
    ckv = gath[:, :D_CKV].reshape(B, K_TS, D_CKV)
    sp = gath[:, D_CKV:D_CKV + D_ROPE].reshape(B, K_TS, D_ROPE)
    mu1 = sp[:, :, :HALF]
    mu2 = sp[:, :, HALF:]
    scos = scos_ref[...][None, :, :]
    ssin = ssin_ref[...][None, :, :]
    kr1 = mu1 * scos - mu2 * ssin
    kr2 = mu1 * ssin + mu2 * scos
    acc = jnp.zeros((B, D_MODEL), dtype=_F32)
    for h in range(N_HEAD):
        qa = qabs_ref[:, h * D_CKV:(h + 1) * D_CKV][:, None, :]
        sc = jnp.sum(qa * ckv, axis=-1)
        qr = qrope_ref[:, h * D_ROPE:(h + 1) * D_ROPE]
        qr1 = qr[:, :HALF][:, None, :]
        qr2 = qr[:, HALF:][:, None, :]
        sr = jnp.sum(qr1 * kr1 + qr2 * kr2, axis=-1)
        s = (sc + sr) * SCALE
        p = jnp.exp(s - jnp.max(s, axis=1, keepdims=True))
        a = p / jnp.sum(p, axis=1, keepdims=True)
        lat = jnp.sum(a[:, :, None] * ckv, axis=1)
        ho = jnp.dot(lat, w_uv_ref[:, h * D_HEAD:(h + 1) * D_HEAD],
                     preferred_element_type=_F32)
        acc = acc + jnp.dot(ho, w_out_ref[h * D_HEAD:(h + 1) * D_HEAD, :],
                            preferred_element_type=_F32)
    out_ref[...] = acc


def _attention(qabs, qrope, gath, scos, ssin, w_uv, w_out):
    B = 128
    d = D_TAB
    return pl.pallas_call(
        _attn_body,
        grid=(L // B,),
        in_specs=[
            pl.BlockSpec((B, N_HEAD * D_CKV), lambda i: (i, 0)),
            pl.BlockSpec((B, N_HEAD * D_ROPE), lambda i: (i, 0)),
            pl.BlockSpec((B * K_TS, d), lambda i: (i, 0)),
            pl.BlockSpec((K_TS, HALF), lambda i: (0, 0)),
            pl.BlockSpec((K_TS, HALF), lambda i: (0, 0)),
            pl.BlockSpec((D_CKV, N_HEAD * D_HEAD), lambda i: (0, 0)),
            pl.BlockSpec((N_HEAD * D_HEAD, D_MODEL), lambda i: (0, 0)),
        ],
        out_specs=pl.BlockSpec((B, D_MODEL), lambda i: (i, 0)),
        out_shape=jax.ShapeDtypeStruct((L, D_MODEL), _F32),
    )(qabs, qrope, gath, scos, ssin, w_uv, w_out)


def kernel(x, w_dkv, w_uk, w_uv, w_dq, w_uq, w_qr, w_kr, raw_delta, w_out,
           idx_wq, idx_wk, idx_w):
    b = x.shape[0]
    x2 = x.reshape(L, D_MODEL)
    theta = 1.0 / (ROPE_BASE ** (2.0 * jnp.arange(HALF, dtype=_F32) / D_ROPE))
    delta = -2.0 * jnp.pi * jax.nn.sigmoid(raw_delta)
    ang_q = jnp.arange(L, dtype=_F32)[:, None] * theta[None, :] + delta[None, :]
    ang_s = jnp.arange(K_TS, dtype=_F32)[:, None] * theta[None, :] + delta[None, :]
    qcos, qsin = jnp.cos(ang_q), jnp.sin(ang_q)
    scos, ssin = jnp.cos(ang_s), jnp.sin(ang_s)

    ckvsp, qabs, qrope, qi, ki = _precompute(
        x2, w_dkv, w_kr, w_dq, w_uq, w_qr, w_uk, idx_wq, idx_wk, qcos, qsin)
    idx = _topk(qi, ki, idx_w.reshape(1, N_IDX))
    gath = _sc_gather(ckvsp, idx.reshape(-1))
    out2 = _attention(qabs, qrope, gath, scos, ssin, w_uv, w_out)
    return out2.reshape(b, L, D_MODEL)


# speedup vs baseline: 1.5052x; 1.0857x over previous
import functools

import jax
import jax.numpy as jnp
import numpy as np
from jax import lax
from jax.experimental import pallas as pl
from jax.experimental.pallas import tpu as pltpu
from jax.experimental.pallas import tpu_sc as plsc

L = 2048
D_MODEL = 768
D_CKV = 192
D_CQ = 256
N_HEAD = 8
D_HEAD = 64
D_ROPE = 32
HALF = D_ROPE // 2
K_TS = 32
WINDOW = 16
N_IDX = 2
ROPE_BASE = 10000.0
SCALE = (D_HEAD + D_ROPE) ** (-0.5)

D_TAB = 256

_F32 = jnp.float32
_NEG_SENTINEL = np.int32(-(2 ** 31))


def _softplus(x):
    return jnp.maximum(x, 0.0) + jnp.log1p(jnp.exp(-jnp.abs(x)))


def _precompute_body(x_ref, w_dkv_ref, w_kr_ref, w_dq_ref, w_uq_ref, w_qr_ref,
                     w_uk_ref, idx_wq_ref, idx_wk_ref, qcos_ref, qsin_ref,
                     ckvsp_ref, qabs_ref, qrope_ref, qi_ref, ki_ref):
    x = x_ref[...]
    ckvsp_ref[:, :D_CKV] = jnp.dot(x, w_dkv_ref[...], preferred_element_type=_F32)
    ckvsp_ref[:, D_CKV:D_CKV + D_ROPE] = _softplus(
        jnp.dot(x, w_kr_ref[...], preferred_element_type=_F32))
    ckvsp_ref[:, D_CKV + D_ROPE:] = jnp.zeros(
        (x.shape[0], D_TAB - D_CKV - D_ROPE), dtype=_F32)
    c_q = jnp.dot(x, w_dq_ref[...], preferred_element_type=_F32)
    q_c = jnp.dot(c_q, w_uq_ref[...], preferred_element_type=_F32)
    for h in range(N_HEAD):
        qc_h = q_c[:, h * D_HEAD:(h + 1) * D_HEAD]
        w_uk_h = w_uk_ref[:, h * D_HEAD:(h + 1) * D_HEAD]
        qabs_ref[:, h * D_CKV:(h + 1) * D_CKV] = lax.dot_general(
            qc_h, w_uk_h, (((1,), (1,)), ((), ())),
            preferred_element_type=_F32)
    sp_q = _softplus(jnp.dot(c_q, w_qr_ref[...], preferred_element_type=_F32))
    qcos = qcos_ref[...]
    qsin = qsin_ref[...]
    for h in range(N_HEAD):
        mu1 = sp_q[:, h * D_ROPE:h * D_ROPE + HALF]
        mu2 = sp_q[:, h * D_ROPE + HALF:(h + 1) * D_ROPE]
        qrope_ref[:, h * D_ROPE:h * D_ROPE + HALF] = mu1 * qcos - mu2 * qsin
        qrope_ref[:, h * D_ROPE + HALF:(h + 1) * D_ROPE] = mu1 * qsin + mu2 * qcos
    qi_ref[...] = jnp.dot(x, idx_wq_ref[...], preferred_element_type=_F32)
    ki_ref[...] = jnp.dot(x, idx_wk_ref[...], preferred_element_type=_F32)


def _precompute(x2, w_dkv, w_kr, w_dq, w_uq, w_qr, w_uk, idx_wq, idx_wk,
                qcos, qsin):
    B = 256
    grid = (L // B,)
    full = lambda shape: pl.BlockSpec(shape, lambda i: (0, 0))
    blk = lambda cols: pl.BlockSpec((B, cols), lambda i: (i, 0))
    return pl.pallas_call(
        _precompute_body,
        grid=grid,
        in_specs=[
            blk(D_MODEL),
            full((D_MODEL, D_CKV)),
            full((D_MODEL, D_ROPE)),
            full((D_MODEL, D_CQ)),
            full((D_CQ, N_HEAD * D_HEAD)),
            full((D_CQ, N_HEAD * D_ROPE)),
            full((D_CKV, N_HEAD * D_HEAD)),
            full((D_MODEL, N_IDX * D_HEAD)),
            full((D_MODEL, D_HEAD)),
            blk(HALF),
            blk(HALF),
        ],
        out_specs=[
            blk(D_TAB),
            blk(N_HEAD * D_CKV),
            blk(N_HEAD * D_ROPE),
            blk(N_IDX * D_HEAD),
            blk(D_HEAD),
        ],
        out_shape=[
            jax.ShapeDtypeStruct((L, D_TAB), _F32),
            jax.ShapeDtypeStruct((L, N_HEAD * D_CKV), _F32),
            jax.ShapeDtypeStruct((L, N_HEAD * D_ROPE), _F32),
            jax.ShapeDtypeStruct((L, N_IDX * D_HEAD), _F32),
            jax.ShapeDtypeStruct((L, D_HEAD), _F32),
        ],
    )(x2, w_dkv, w_kr, w_dq, w_uq, w_qr, w_uk, idx_wq, idx_wk, qcos, qsin)


def _topk_body(qi_ref, ki_ref, idxw_ref, idx_ref, keys_ref):
    blk = pl.program_id(0)
    B = qi_ref.shape[0]
    ki = ki_ref[...]
    acc = None
    for h in range(N_IDX):
        qi_h = qi_ref[:, h * D_HEAD:(h + 1) * D_HEAD]
        s = lax.dot_general(qi_h, ki, (((1,), (1,)), ((), ())),
                            preferred_element_type=_F32)
        s = s.astype(jnp.bfloat16).astype(_F32)
        s = jnp.maximum(s, 0.0) * idxw_ref[0, h]
        acc = s if acc is None else acc + s
    acc = jnp.where(acc == 0.0, 0.0, acc)
    row = blk * B + lax.broadcasted_iota(jnp.int32, (B, L), 0)
    col = lax.broadcasted_iota(jnp.int32, (B, L), 1)
    local = (col >= row - (WINDOW - 1)) & (col <= row)
    future = col > row
    vals = jnp.where(local, jnp.inf, jnp.where(future, -jnp.inf, acc))
    keys_ref[...] = lax.bitcast_convert_type(vals, jnp.int32)
    cols = []
    for _ in range(K_TS):
        k = keys_ref[...]
        m = jnp.max(k, axis=1, keepdims=True)
        cand = jnp.where(k == m, col, jnp.int32(L))
        amin = jnp.min(cand, axis=1)
        cols.append(amin.reshape(B, 1))
        keys_ref[...] = jnp.where(col == amin[:, None], _NEG_SENTINEL, k)
    idx_ref[...] = jnp.concatenate(cols, axis=1)


def _topk_fast_body(qi_ref, ki_ref, idxw_ref, idx_ref, keys_ref):
    blk = pl.program_id(0) + 1
    B = qi_ref.shape[0]
    ki = ki_ref[...]
    acc = None
    for h in range(N_IDX):
        qi_h = qi_ref[:, h * D_HEAD:(h + 1) * D_HEAD]
        s = lax.dot_general(qi_h, ki, (((1,), (1,)), ((), ())),
                            preferred_element_type=_F32)
        s = s.astype(jnp.bfloat16).astype(_F32)
        s = jnp.maximum(s, 0.0) * idxw_ref[0, h]
        acc = s if acc is None else acc + s
    acc = jnp.where(acc == 0.0, 0.0, acc)
    row = blk * B + lax.broadcasted_iota(jnp.int32, (B, L), 0)
    col = lax.broadcasted_iota(jnp.int32, (B, L), 1)
    keys0 = lax.bitcast_convert_type(acc, jnp.int32)
    keys_ref[...] = jnp.where(col <= row - WINDOW, keys0, _NEG_SENTINEL)
    wcol = lax.broadcasted_iota(jnp.int32, (B, WINDOW), 1)
    wrow = blk * B + lax.broadcasted_iota(jnp.int32, (B, WINDOW), 0)
    cols = [wrow - (WINDOW - 1) + wcol]
    for _ in range(K_TS - WINDOW):
        k = keys_ref[...]
        m = jnp.max(k, axis=1, keepdims=True)
        cand = jnp.where(k == m, col, jnp.int32(L))
        amin = jnp.min(cand, axis=1)
        cols.append(amin.reshape(B, 1))
        keys_ref[...] = jnp.where(col == amin[:, None], _NEG_SENTINEL, k)
    idx_ref[...] = jnp.concatenate(cols, axis=1)


def _topk(qi, ki, idx_w2):
    B = 256
    idx0 = pl.pallas_call(
        _topk_body,
        grid=(1,),
        in_specs=[
            pl.BlockSpec((B, N_IDX * D_HEAD), lambda i: (i, 0)),
            pl.BlockSpec((L, D_HEAD), lambda i: (0, 0)),
            pl.BlockSpec(memory_space=pltpu.SMEM),
        ],
        out_specs=pl.BlockSpec((B, K_TS), lambda i: (i, 0)),
        out_shape=jax.ShapeDtypeStruct((B, K_TS), jnp.int32),
        scratch_shapes=[pltpu.VMEM((B, L), jnp.int32)],
    )(qi, ki, idx_w2)
    idx1 = pl.pallas_call(
        _topk_fast_body,
        grid=(L // B - 1,),
        in_specs=[
            pl.BlockSpec((B, N_IDX * D_HEAD), lambda i: (i + 1, 0)),
            pl.BlockSpec((L, D_HEAD), lambda i: (0, 0)),
            pl.BlockSpec(memory_space=pltpu.SMEM),
        ],
        out_specs=pl.BlockSpec((B, K_TS), lambda i: (i, 0)),
        out_shape=jax.ShapeDtypeStruct((L - B, K_TS), jnp.int32),
        scratch_shapes=[pltpu.VMEM((B, L), jnp.int32)],
    )(qi, ki, idx_w2)
    return jnp.concatenate([idx0, idx1], axis=0)


def _sc_gather(table, idx_flat):
    n_rows = L * K_TS
    d = D_TAB
    info = plsc.get_sparse_core_info()
    nw = info.num_cores * info.num_subcores
    chunk = 128
    per_w = n_rows // nw
    n_chunks = per_w // chunk
    mesh = plsc.VectorSubcoreMesh(core_axis_name="c", subcore_axis_name="s")

    @functools.partial(
        pl.kernel,
        mesh=mesh,
        out_type=jax.ShapeDtypeStruct((n_rows, d), _F32),
        scratch_types=[
            pltpu.VMEM((chunk,), jnp.int32),
            pltpu.VMEM((chunk, d), _F32),
            pltpu.SemaphoreType.DMA,
        ],
    )
    def gather_k(table_hbm, idx_hbm, out_hbm, idx_v, rows_v, sem):
        wid = lax.axis_index("s") * info.num_cores + lax.axis_index("c")

        def body(i, _):
            base = wid * per_w + i * chunk
            pltpu.sync_copy(idx_hbm.at[pl.ds(base, chunk)], idx_v)
            pltpu.async_copy(table_hbm.at[idx_v], rows_v, sem).wait()
            pltpu.sync_copy(rows_v, out_hbm.at[pl.ds(base, chunk)])
            return 0

        lax.fori_loop(0, n_chunks, body, 0)

    return gather_k(table, idx_flat)


def _attn_body(qabs_ref, qrope_ref, gath_ref, scos_ref, ssin_ref,
               w_uv_ref, w_out_ref, out_ref):
    B = qabs_ref.shape[0]
    gath = gath_ref[...]
    ckv = gath[:, :D_CKV].reshape(B, K_TS, D_CKV)
    sp = gath[:, D_CKV:D_CKV + D_ROPE].reshape(B, K_TS, D_ROPE)
    mu1 = sp[:, :, :HALF]
    mu2 = sp[:, :, HALF:]
    scos = scos_ref[...][None, :, :]
    ssin = ssin_ref[...][None, :, :]
    kr1 = mu1 * scos - mu2 * ssin
    kr2 = mu1 * ssin + mu2 * scos
    acc = jnp.zeros((B, D_MODEL), dtype=_F32)
    for h in range(N_HEAD):
        qa = qabs_ref[:, h * D_CKV:(h + 1) * D_CKV][:, None, :]
        sc = jnp.sum(qa * ckv, axis=-1)
        qr = qrope_ref[:, h * D_ROPE:(h + 1) * D_ROPE]
        qr1 = qr[:, :HALF][:, None, :]
        qr2 = qr[:, HALF:][:, None, :]
        sr = jnp.sum(qr1 * kr1 + qr2 * kr2, axis=-1)
        s = (sc + sr) * SCALE
        p = jnp.exp(s - jnp.max(s, axis=1, keepdims=True))
        a = p / jnp.sum(p, axis=1, keepdims=True)
        lat = jnp.sum(a[:, :, None] * ckv, axis=1)
        ho = jnp.dot(lat, w_uv_ref[:, h * D_HEAD:(h + 1) * D_HEAD],
                     preferred_element_type=_F32)
        acc = acc + jnp.dot(ho, w_out_ref[h * D_HEAD:(h + 1) * D_HEAD, :],
                            preferred_element_type=_F32)
    out_ref[...] = acc


def _attention(qabs, qrope, gath, scos, ssin, w_uv, w_out):
    B = 128
    d = D_TAB
    return pl.pallas_call(
        _attn_body,
        grid=(L // B,),
        in_specs=[
            pl.BlockSpec((B, N_HEAD * D_CKV), lambda i: (i, 0)),
            pl.BlockSpec((B, N_HEAD * D_ROPE), lambda i: (i, 0)),
            pl.BlockSpec((B * K_TS, d), lambda i: (i, 0)),
            pl.BlockSpec((K_TS, HALF), lambda i: (0, 0)),
            pl.BlockSpec((K_TS, HALF), lambda i: (0, 0)),
            pl.BlockSpec((D_CKV, N_HEAD * D_HEAD), lambda i: (0, 0)),
            pl.BlockSpec((N_HEAD * D_HEAD, D_MODEL), lambda i: (0, 0)),
        ],
        out_specs=pl.BlockSpec((B, D_MODEL), lambda i: (i, 0)),
        out_shape=jax.ShapeDtypeStruct((L, D_MODEL), _F32),
    )(qabs, qrope, gath, scos, ssin, w_uv, w_out)


def kernel(x, w_dkv, w_uk, w_uv, w_dq, w_uq, w_qr, w_kr, raw_delta, w_out,
           idx_wq, idx_wk, idx_w):
    b = x.shape[0]
    x2 = x.reshape(L, D_MODEL)
    theta = 1.0 / (ROPE_BASE ** (2.0 * jnp.arange(HALF, dtype=_F32) / D_ROPE))
    delta = -2.0 * jnp.pi * jax.nn.sigmoid(raw_delta)
    ang_q = jnp.arange(L, dtype=_F32)[:, None] * theta[None, :] + delta[None, :]
    ang_s = jnp.arange(K_TS, dtype=_F32)[:, None] * theta[None, :] + delta[None, :]
    qcos, qsin = jnp.cos(ang_q), jnp.sin(ang_q)
    scos, ssin = jnp.cos(ang_s), jnp.sin(ang_s)

    ckvsp, qabs, qrope, qi, ki = _precompute(
        x2, w_dkv, w_kr, w_dq, w_uq, w_qr, w_uk, idx_wq, idx_wk, qcos, qsin)
    idx = _topk(qi, ki, idx_w.reshape(1, N_IDX))
    gath = _sc_gather(ckvsp, idx.reshape(-1))
    out2 = _attention(qabs, qrope, gath, scos, ssin, w_uv, w_out)
    return out2.reshape(b, L, D_MODEL)


# speedup vs baseline: 5.0417x; 3.3495x over previous
import functools

import jax
import jax.numpy as jnp
import numpy as np
from jax import lax
from jax.experimental import pallas as pl
from jax.experimental.pallas import tpu as pltpu
from jax.experimental.pallas import tpu_sc as plsc

L = 2048
D_MODEL = 768
D_CKV = 192
D_CQ = 256
N_HEAD = 8
D_HEAD = 64
D_ROPE = 32
HALF = D_ROPE // 2
K_TS = 32
WINDOW = 16
N_IDX = 2
ROPE_BASE = 10000.0
SCALE = (D_HEAD + D_ROPE) ** (-0.5)

D_TAB = 256

_F32 = jnp.float32
_NEG_SENTINEL = np.int32(-(2 ** 31))


def _softplus(x):
    return jnp.maximum(x, 0.0) + jnp.log1p(jnp.exp(-jnp.abs(x)))


def _precompute_body(x_ref, w_dkv_ref, w_kr_ref, w_dq_ref, w_uq_ref, w_qr_ref,
                     w_uk_ref, idx_wq_ref, idx_wk_ref, qcos_ref, qsin_ref,
                     ckvsp_ref, qabs_ref, qrope_ref, qi_ref, ki_ref):
    x = x_ref[...]
    ckvsp_ref[:, :D_CKV] = jnp.dot(x, w_dkv_ref[...], preferred_element_type=_F32)
    ckvsp_ref[:, D_CKV:D_CKV + D_ROPE] = _softplus(
        jnp.dot(x, w_kr_ref[...], preferred_element_type=_F32))
    ckvsp_ref[:, D_CKV + D_ROPE:] = jnp.zeros(
        (x.shape[0], D_TAB - D_CKV - D_ROPE), dtype=_F32)
    c_q = jnp.dot(x, w_dq_ref[...], preferred_element_type=_F32)
    q_c = jnp.dot(c_q, w_uq_ref[...], preferred_element_type=_F32)
    for h in range(N_HEAD):
        qc_h = q_c[:, h * D_HEAD:(h + 1) * D_HEAD]
        w_uk_h = w_uk_ref[:, h * D_HEAD:(h + 1) * D_HEAD]
        qabs_ref[:, h * D_CKV:(h + 1) * D_CKV] = lax.dot_general(
            qc_h, w_uk_h, (((1,), (1,)), ((), ())),
            preferred_element_type=_F32)
    sp_q = _softplus(jnp.dot(c_q, w_qr_ref[...], preferred_element_type=_F32))
    qcos = qcos_ref[...]
    qsin = qsin_ref[...]
    for h in range(N_HEAD):
        mu1 = sp_q[:, h * D_ROPE:h * D_ROPE + HALF]
        mu2 = sp_q[:, h * D_ROPE + HALF:(h + 1) * D_ROPE]
        qrope_ref[:, h * D_ROPE:h * D_ROPE + HALF] = mu1 * qcos - mu2 * qsin
        qrope_ref[:, h * D_ROPE + HALF:(h + 1) * D_ROPE] = mu1 * qsin + mu2 * qcos
    qi_ref[...] = jnp.dot(x, idx_wq_ref[...], preferred_element_type=_F32)
    ki_ref[...] = jnp.dot(x, idx_wk_ref[...], preferred_element_type=_F32)


def _precompute(x2, w_dkv, w_kr, w_dq, w_uq, w_qr, w_uk, idx_wq, idx_wk,
                qcos, qsin):
    B = 256
    grid = (L // B,)
    full = lambda shape: pl.BlockSpec(shape, lambda i: (0, 0))
    blk = lambda cols: pl.BlockSpec((B, cols), lambda i: (i, 0))
    return pl.pallas_call(
        _precompute_body,
        grid=grid,
        in_specs=[
            blk(D_MODEL),
            full((D_MODEL, D_CKV)),
            full((D_MODEL, D_ROPE)),
            full((D_MODEL, D_CQ)),
            full((D_CQ, N_HEAD * D_HEAD)),
            full((D_CQ, N_HEAD * D_ROPE)),
            full((D_CKV, N_HEAD * D_HEAD)),
            full((D_MODEL, N_IDX * D_HEAD)),
            full((D_MODEL, D_HEAD)),
            blk(HALF),
            blk(HALF),
        ],
        out_specs=[
            blk(D_TAB),
            blk(N_HEAD * D_CKV),
            blk(N_HEAD * D_ROPE),
            blk(N_IDX * D_HEAD),
            blk(D_HEAD),
        ],
        out_shape=[
            jax.ShapeDtypeStruct((L, D_TAB), _F32),
            jax.ShapeDtypeStruct((L, N_HEAD * D_CKV), _F32),
            jax.ShapeDtypeStruct((L, N_HEAD * D_ROPE), _F32),
            jax.ShapeDtypeStruct((L, N_IDX * D_HEAD), _F32),
            jax.ShapeDtypeStruct((L, D_HEAD), _F32),
        ],
    )(x2, w_dkv, w_kr, w_dq, w_uq, w_qr, w_uk, idx_wq, idx_wk, qcos, qsin)


def _topk_body(qi_ref, ki_ref, idxw_ref, idx_ref, keys_ref):
    blk = pl.program_id(0)
    B = qi_ref.shape[0]
    ki = ki_ref[...]
    acc = None
    for h in range(N_IDX):
        qi_h = qi_ref[:, h * D_HEAD:(h + 1) * D_HEAD]
        s = lax.dot_general(qi_h, ki, (((1,), (1,)), ((), ())),
                            preferred_element_type=_F32)
        s = s.astype(jnp.bfloat16).astype(_F32)
        s = jnp.maximum(s, 0.0) * idxw_ref[0, h]
        acc = s if acc is None else acc + s
    acc = jnp.where(acc == 0.0, 0.0, acc)
    row = blk * B + lax.broadcasted_iota(jnp.int32, (B, L), 0)
    col = lax.broadcasted_iota(jnp.int32, (B, L), 1)
    local = (col >= row - (WINDOW - 1)) & (col <= row)
    future = col > row
    vals = jnp.where(local, jnp.inf, jnp.where(future, -jnp.inf, acc))
    keys_ref[...] = lax.bitcast_convert_type(vals, jnp.int32)
    cols = []
    for _ in range(K_TS):
        k = keys_ref[...]
        m = jnp.max(k, axis=1, keepdims=True)
        cand = jnp.where(k == m, col, jnp.int32(L))
        amin = jnp.min(cand, axis=1)
        cols.append(amin.reshape(B, 1))
        keys_ref[...] = jnp.where(col == amin[:, None], _NEG_SENTINEL, k)
    idx_ref[...] = jnp.concatenate(cols, axis=1)


def _topk_fast_body(qi_ref, ki_ref, idxw_ref, idx_ref, keys_ref):
    blk = pl.program_id(0) + 1
    B = qi_ref.shape[0]
    ki = ki_ref[...]
    acc = None
    for h in range(N_IDX):
        qi_h = qi_ref[:, h * D_HEAD:(h + 1) * D_HEAD]
        s = lax.dot_general(qi_h, ki, (((1,), (1,)), ((), ())),
                            preferred_element_type=_F32)
        s = s.astype(jnp.bfloat16).astype(_F32)
        s = jnp.maximum(s, 0.0) * idxw_ref[0, h]
        acc = s if acc is None else acc + s
    acc = jnp.where(acc == 0.0, 0.0, acc)
    row = blk * B + lax.broadcasted_iota(jnp.int32, (B, L), 0)
    col = lax.broadcasted_iota(jnp.int32, (B, L), 1)
    keys0 = lax.bitcast_convert_type(acc, jnp.int32)
    keys_ref[...] = jnp.where(col <= row - WINDOW, keys0, _NEG_SENTINEL)
    wcol = lax.broadcasted_iota(jnp.int32, (B, WINDOW), 1)
    wrow = blk * B + lax.broadcasted_iota(jnp.int32, (B, WINDOW), 0)
    cols = [wrow - (WINDOW - 1) + wcol]
    for _ in range(K_TS - WINDOW):
        k = keys_ref[...]
        m = jnp.max(k, axis=1, keepdims=True)
        cand = jnp.where(k == m, col, jnp.int32(L))
        amin = jnp.min(cand, axis=1)
        cols.append(amin.reshape(B, 1))
        keys_ref[...] = jnp.where(col == amin[:, None], _NEG_SENTINEL, k)
    idx_ref[...] = jnp.concatenate(cols, axis=1)


def _topk(qi, ki, idx_w2):
    B = 256
    idx0 = pl.pallas_call(
        _topk_body,
        grid=(1,),
        in_specs=[
            pl.BlockSpec((B, N_IDX * D_HEAD), lambda i: (i, 0)),
            pl.BlockSpec((L, D_HEAD), lambda i: (0, 0)),
            pl.BlockSpec(memory_space=pltpu.SMEM),
        ],
        out_specs=pl.BlockSpec((B, K_TS), lambda i: (i, 0)),
        out_shape=jax.ShapeDtypeStruct((B, K_TS), jnp.int32),
        scratch_shapes=[pltpu.VMEM((B, L), jnp.int32)],
    )(qi, ki, idx_w2)
    idx1 = pl.pallas_call(
        _topk_fast_body,
        grid=(L // B - 1,),
        in_specs=[
            pl.BlockSpec((B, N_IDX * D_HEAD), lambda i: (i + 1, 0)),
            pl.BlockSpec((L, D_HEAD), lambda i: (0, 0)),
            pl.BlockSpec(memory_space=pltpu.SMEM),
        ],
        out_specs=pl.BlockSpec((B, K_TS), lambda i: (i, 0)),
        out_shape=jax.ShapeDtypeStruct((L - B, K_TS), jnp.int32),
        scratch_shapes=[pltpu.VMEM((B, L), jnp.int32)],
    )(qi, ki, idx_w2)
    return jnp.concatenate([idx0, idx1], axis=0)


def _sc_gather(table, idx_flat):
    n_rows = L * K_TS
    d = D_TAB
    info = plsc.get_sparse_core_info()
    nw = info.num_cores * info.num_subcores
    chunk = 128
    per_w = n_rows // nw
    n_chunks = per_w // chunk
    mesh = plsc.VectorSubcoreMesh(core_axis_name="c", subcore_axis_name="s")

    @functools.partial(
        pl.kernel,
        mesh=mesh,
        out_type=jax.ShapeDtypeStruct((n_rows, d), _F32),
        scratch_types=[
            pltpu.VMEM((chunk,), jnp.int32),
            pltpu.VMEM((chunk, d), _F32),
            pltpu.SemaphoreType.DMA,
        ],
    )
    def gather_k(table_hbm, idx_hbm, out_hbm, idx_v, rows_v, sem):
        wid = lax.axis_index("s") * info.num_cores + lax.axis_index("c")

        def body(i, _):
            base = wid * per_w + i * chunk
            pltpu.sync_copy(idx_hbm.at[pl.ds(base, chunk)], idx_v)
            pltpu.async_copy(table_hbm.at[idx_v], rows_v, sem).wait()
            pltpu.sync_copy(rows_v, out_hbm.at[pl.ds(base, chunk)])
            return 0

        lax.fori_loop(0, n_chunks, body, 0)

    return gather_k(table, idx_flat)


def _attn_body(qabs_ref, qrope_ref, gath_ref, scos_ref, ssin_ref,
               w_uv_ref, w_out_ref, out_ref):
    B = qabs_ref.shape[0]
    gath = gath_ref[...]
    ckv = gath[:, :D_CKV].reshape(B, K_TS, D_CKV)
    sp = gath[:, D_CKV:D_CKV + D_ROPE].reshape(B, K_TS, D_ROPE)
    mu1 = sp[:, :, :HALF]
    mu2 = sp[:, :, HALF:]
    scos = scos_ref[...][None, :, :]
    ssin = ssin_ref[...][None, :, :]
    kr1 = mu1 * scos - mu2 * ssin
    kr2 = mu1 * ssin + mu2 * scos
    kr = jnp.concatenate([kr1, kr2], axis=-1)
    qabs3 = qabs_ref[...].reshape(B, N_HEAD, D_CKV)
    qr3 = qrope_ref[...].reshape(B, N_HEAD, D_ROPE)
    bdims = (((2,), (2,)), ((0,), (0,)))
    sc = lax.dot_general(qabs3, ckv, bdims, preferred_element_type=_F32)
    sr = lax.dot_general(qr3, kr, bdims, preferred_element_type=_F32)
    s = (sc + sr) * SCALE
    p = jnp.exp(s - jnp.max(s, axis=2, keepdims=True))
    a = p / jnp.sum(p, axis=2, keepdims=True)
    lat = lax.dot_general(a, ckv, (((2,), (1,)), ((0,), (0,))),
                          preferred_element_type=_F32)
    acc = jnp.zeros((B, D_MODEL), dtype=_F32)
    for h in range(N_HEAD):
        ho = jnp.dot(lat[:, h, :], w_uv_ref[:, h * D_HEAD:(h + 1) * D_HEAD],
                     preferred_element_type=_F32)
        acc = acc + jnp.dot(ho, w_out_ref[h * D_HEAD:(h + 1) * D_HEAD, :],
                            preferred_element_type=_F32)
    out_ref[...] = acc


def _attention(qabs, qrope, gath, scos, ssin, w_uv, w_out):
    B = 128
    d = D_TAB
    return pl.pallas_call(
        _attn_body,
        grid=(L // B,),
        in_specs=[
            pl.BlockSpec((B, N_HEAD * D_CKV), lambda i: (i, 0)),
            pl.BlockSpec((B, N_HEAD * D_ROPE), lambda i: (i, 0)),
            pl.BlockSpec((B * K_TS, d), lambda i: (i, 0)),
            pl.BlockSpec((K_TS, HALF), lambda i: (0, 0)),
            pl.BlockSpec((K_TS, HALF), lambda i: (0, 0)),
            pl.BlockSpec((D_CKV, N_HEAD * D_HEAD), lambda i: (0, 0)),
            pl.BlockSpec((N_HEAD * D_HEAD, D_MODEL), lambda i: (0, 0)),
        ],
        out_specs=pl.BlockSpec((B, D_MODEL), lambda i: (i, 0)),
        out_shape=jax.ShapeDtypeStruct((L, D_MODEL), _F32),
    )(qabs, qrope, gath, scos, ssin, w_uv, w_out)


def kernel(x, w_dkv, w_uk, w_uv, w_dq, w_uq, w_qr, w_kr, raw_delta, w_out,
           idx_wq, idx_wk, idx_w):
    b = x.shape[0]
    x2 = x.reshape(L, D_MODEL)
    theta = 1.0 / (ROPE_BASE ** (2.0 * jnp.arange(HALF, dtype=_F32) / D_ROPE))
    delta = -2.0 * jnp.pi * jax.nn.sigmoid(raw_delta)
    ang_q = jnp.arange(L, dtype=_F32)[:, None] * theta[None, :] + delta[None, :]
    ang_s = jnp.arange(K_TS, dtype=_F32)[:, None] * theta[None, :] + delta[None, :]
    qcos, qsin = jnp.cos(ang_q), jnp.sin(ang_q)
    scos, ssin = jnp.cos(ang_s), jnp.sin(ang_s)

    ckvsp, qabs, qrope, qi, ki = _precompute(
        x2, w_dkv, w_kr, w_dq, w_uq, w_qr, w_uk, idx_wq, idx_wk, qcos, qsin)
    idx = _topk(qi, ki, idx_w.reshape(1, N_IDX))
    gath = _sc_gather(ckvsp, idx.reshape(-1))
    out2 = _attention(qabs, qrope, gath, scos, ssin, w_uv, w_out)
    return out2.reshape(b, L, D_MODEL)
